# baseline (speedup 1.0000x reference)
import functools

import jax
import jax.numpy as jnp
from jax import lax
from jax.experimental import pallas as pl
from jax.experimental.pallas import tpu as pltpu
from jax.experimental.pallas import tpu_sc as plsc

NC = 2
NS = 16
LANES = 16
NW = NC * NS


@functools.partial(jax.jit, static_argnames=("B", "V", "A", "c_pad"))
def _sc_pool(flat_logits, idx_flat, w_flat, *, B, V, A, c_pad):
    n_idx = A * c_pad
    rows_per_w = B // NW
    cblocks = c_pad // LANES

    mesh = plsc.VectorSubcoreMesh(
        core_axis_name="c", subcore_axis_name="s", num_cores=NC, num_subcores=NS
    )

    @functools.partial(
        pl.kernel,
        mesh=mesh,
        out_type=jax.ShapeDtypeStruct((B, c_pad), jnp.float32),
        scratch_types=[
            pltpu.VMEM((n_idx,), jnp.int32),
            pltpu.VMEM((n_idx,), jnp.float32),
            pltpu.VMEM((n_idx,), jnp.int32),
            pltpu.VMEM((n_idx,), jnp.float32),
            pltpu.VMEM((rows_per_w, c_pad), jnp.float32),
            pltpu.SemaphoreType.DMA,
        ],
    )
    def pool(flat_hbm, idx_hbm, w_hbm, out_hbm, idx_v, w_v, idxr_v, g_v, acc_v, sem):
        wid = lax.axis_index("s") * NC + lax.axis_index("c")
        pltpu.sync_copy(idx_hbm, idx_v)
        pltpu.sync_copy(w_hbm, w_v)
        row0 = wid * rows_per_w

        def row_body(r, carry):
            shift = (row0 + r) * V

            def shift_blk(k, c2):
                off = k * LANES
                idxr_v[pl.ds(off, LANES)] = idx_v[pl.ds(off, LANES)] + shift
                return c2

            lax.fori_loop(0, n_idx // LANES, shift_blk, 0)
            pltpu.async_copy(flat_hbm.at[idxr_v], g_v, sem).wait()

            for cb in range(cblocks):
                def pool_a(a, acc):
                    off = a * c_pad + cb * LANES
                    return acc + g_v[pl.ds(off, LANES)] * w_v[pl.ds(off, LANES)]

                acc = lax.fori_loop(0, A, pool_a, jnp.zeros((LANES,), jnp.float32))
                acc_v[r, pl.ds(cb * LANES, LANES)] = acc
            return carry

        lax.fori_loop(0, rows_per_w, row_body, 0)
        pltpu.sync_copy(acc_v, out_hbm.at[pl.ds(row0, rows_per_w)])

    return pool(flat_logits, idx_flat, w_flat)


def _norm_combine_body(x_ref, pooled_ref, mask_ref, out_ref, m_ref, s_ref, *,
                       nv, vb, V, C):
    j = pl.program_id(1)

    @pl.when(j == 0)
    def _init():
        m_ref[...] = jnp.full_like(m_ref, -jnp.inf)
        s_ref[...] = jnp.zeros_like(s_ref)

    rem = V - (nv - 1) * vb
    if rem < vb:
        @pl.when(j == nv - 1)
        def _mask_tail():
            x_ref[:, rem:] = jnp.full_like(x_ref[:, rem:], -jnp.inf)

    x = x_ref[...]
    bm = jnp.max(x, axis=1, keepdims=True)
    m_old = m_ref[...]
    m_new = jnp.maximum(m_old, bm)
    alpha = jnp.where(m_old == m_new, 1.0, jnp.exp(m_old - m_new))
    e = jnp.exp(x - m_new)
    s_new = s_ref[...] * alpha + jnp.sum(e, axis=1, keepdims=True)
    m_ref[...] = m_new
    s_ref[...] = s_new

    @pl.when(j == nv - 1)
    def _finalize():
        norm = m_ref[...] + jnp.log(s_ref[...])
        mask = mask_ref[...]
        summask = jnp.sum(mask, axis=0, keepdims=True)
        denom = jnp.clip(summask, 1e-9, None)
        inv = 1.0 / denom
        scale = summask * inv
        pooled = pooled_ref[...]
        res = pooled * inv - norm * scale
        out_ref[...] = res[:, :C]


@functools.partial(jax.jit, static_argnames=("C", "Bb", "vb"))
def _norm_combine(logits, pooled, mask_t, *, C, Bb=256, vb=8192):
    B, V = logits.shape
    A, c_pad = mask_t.shape
    nb = B // Bb
    nv = -(-V // vb)

    return pl.pallas_call(
        functools.partial(_norm_combine_body, nv=nv, vb=vb, V=V, C=C),
        grid=(nb, nv),
        in_specs=[
            pl.BlockSpec((Bb, vb), lambda i, j: (i, j)),
            pl.BlockSpec((Bb, c_pad), lambda i, j: (i, 0)),
            pl.BlockSpec((A, c_pad), lambda i, j: (0, 0)),
        ],
        out_specs=pl.BlockSpec((Bb, C), lambda i, j: (i, 0)),
        out_shape=jax.ShapeDtypeStruct((B, C), jnp.float32),
        scratch_shapes=[
            pltpu.VMEM((Bb, 1), jnp.float32),
            pltpu.VMEM((Bb, 1), jnp.float32),
        ],
        compiler_params=pltpu.CompilerParams(
            dimension_semantics=("parallel", "arbitrary"),
        ),
    )(logits, pooled, mask_t)


def kernel(logits, word2label, label_words_mask):
    B, V = logits.shape
    C, A = word2label.shape
    c_pad = -(-C // LANES) * LANES

    idx_t = jnp.zeros((A, c_pad), jnp.int32).at[:, :C].set(word2label.T)
    w_t = jnp.zeros((A, c_pad), jnp.float32).at[:, :C].set(
        label_words_mask.astype(jnp.float32).T
    )

    pooled = _sc_pool(
        logits.reshape(-1), idx_t.reshape(-1), w_t.reshape(-1),
        B=B, V=V, A=A, c_pad=c_pad,
    )
    return _norm_combine(logits, pooled, w_t, C=C)


# speedup vs baseline: 1.8718x; 1.8718x over previous
import functools

import jax
import jax.numpy as jnp
from jax import lax
from jax.experimental import pallas as pl
from jax.experimental.pallas import tpu as pltpu
from jax.experimental.pallas import tpu_sc as plsc

NC = 2
NS = 16
LANES = 16
NW = NC * NS
CP = 128


@functools.partial(jax.jit, static_argnames=("V_S", "A"))
def _sc_build_s(idx_flat, w_flat, *, V_S, A):
    n_ent = A * CP
    rows_per_w = V_S // NW
    chunk = 784
    n_full = rows_per_w // chunk
    sizes = [chunk] * n_full
    if rows_per_w % chunk:
        sizes.append(rows_per_w % chunk)
    vec_per_a = CP // LANES

    mesh = plsc.VectorSubcoreMesh(
        core_axis_name="c", subcore_axis_name="s", num_cores=NC, num_subcores=NS
    )

    @functools.partial(
        pl.kernel,
        mesh=mesh,
        compiler_params=pltpu.CompilerParams(needs_layout_passes=False),
        out_type=jax.ShapeDtypeStruct((V_S * CP,), jnp.float32),
        scratch_types=[
            pltpu.VMEM((n_ent,), jnp.int32),
            pltpu.VMEM((n_ent,), jnp.float32),
            pltpu.VMEM((chunk * CP,), jnp.float32),
        ],
    )
    def build_s(idx_hbm, w_hbm, s_hbm, idx_v, w_v, chunk_v):
        wid = lax.axis_index("s") * NC + lax.axis_index("c")
        pltpu.sync_copy(idx_hbm, idx_v)
        pltpu.sync_copy(w_hbm, w_v)
        row0 = wid * rows_per_w

        zeros16 = jnp.zeros((LANES,), jnp.float32)

        def zvec(q, _):
            chunk_v[pl.ds(q * LANES, LANES)] = zeros16
            return _

        lax.fori_loop(0, chunk * CP // LANES, zvec, 0)

        lane = lax.iota(jnp.int32, LANES)
        off = 0
        for size in sizes:
            base = row0 + off

            def scan(k, base_=base, size_=size, store_zero=False):
                ent = idx_v[pl.ds(k * LANES, LANES)]
                row = ent - base_
                ok = (row >= 0) & (row < size_)
                row = jnp.clip(row, 0, size_ - 1)
                col = (k % vec_per_a) * LANES + lane
                flat = row * CP + col
                if store_zero:
                    plsc.store_scatter(chunk_v, [flat], zeros16, mask=ok)
                else:
                    w = w_v[pl.ds(k * LANES, LANES)]
                    plsc.addupdate_scatter(chunk_v, [flat], w, mask=ok)

            def add_body(k, _):
                scan(k)
                return _

            lax.fori_loop(0, n_ent // LANES, add_body, 0)
            pltpu.sync_copy(
                chunk_v.at[pl.ds(0, size * CP)],
                s_hbm.at[pl.ds(base * CP, size * CP)],
            )

            def rezero_body(k, _):
                scan(k, store_zero=True)
                return _

            lax.fori_loop(0, n_ent // LANES, rezero_body, 0)
            off += size

    return build_s(idx_flat, w_flat).reshape(V_S, CP)


def _lse_mm_body(x_ref, s_ref, mask_ref, out_ref, m_ref, sum_ref, acc_ref, *,
                 nv, vb, V, C):
    j = pl.program_id(0)
    i = pl.program_id(1)

    rem = V - (nv - 1) * vb
    if rem < vb:
        @pl.when(j == nv - 1)
        def _mask_tail():
            x_ref[:, rem:] = jnp.full_like(x_ref[:, rem:], -1e30)

    x = x_ref[...]
    bm = jnp.max(x, axis=1, keepdims=True)
    dot = jnp.dot(x, s_ref[...], preferred_element_type=jnp.float32)

    @pl.when(j == 0)
    def _init():
        m_ref[i] = bm
        sum_ref[i] = jnp.sum(jnp.exp(x - bm), axis=1, keepdims=True)
        acc_ref[i] = dot

    @pl.when(j > 0)
    def _accum():
        m_old = m_ref[i]
        m_new = jnp.maximum(m_old, bm)
        alpha = jnp.where(m_old == m_new, 1.0, jnp.exp(m_old - m_new))
        e = jnp.exp(x - m_new)
        sum_ref[i] = sum_ref[i] * alpha + jnp.sum(e, axis=1, keepdims=True)
        m_ref[i] = m_new
        acc_ref[i] = acc_ref[i] + dot

    @pl.when(j == nv - 1)
    def _finalize():
        norm = m_ref[i] + jnp.log(sum_ref[i])
        mask = mask_ref[...]
        summask = jnp.sum(mask, axis=0, keepdims=True)
        denom = jnp.clip(summask, 1e-9, None)
        inv = 1.0 / denom
        scale = summask * inv
        res = acc_ref[i] * inv - norm * scale
        out_ref[...] = res[:, :C]


@functools.partial(jax.jit, static_argnames=("C", "Bb", "vb"))
def _lse_mm(logits, s_mat, mask_t, *, C, Bb, vb):
    B, V = logits.shape
    V_S, _ = s_mat.shape
    A, _ = mask_t.shape
    nb = B // Bb
    nv = V_S // vb

    return pl.pallas_call(
        functools.partial(_lse_mm_body, nv=nv, vb=vb, V=V, C=C),
        grid=(nv, nb),
        in_specs=[
            pl.BlockSpec((Bb, vb), lambda j, i: (i, j)),
            pl.BlockSpec((vb, CP), lambda j, i: (j, 0)),
            pl.BlockSpec((A, CP), lambda j, i: (0, 0)),
        ],
        out_specs=pl.BlockSpec((Bb, C), lambda j, i: (i, 0)),
        out_shape=jax.ShapeDtypeStruct((B, C), jnp.float32),
        scratch_shapes=[
            pltpu.VMEM((nb, Bb, 1), jnp.float32),
            pltpu.VMEM((nb, Bb, 1), jnp.float32),
            pltpu.VMEM((nb, Bb, CP), jnp.float32),
        ],
        compiler_params=pltpu.CompilerParams(
            dimension_semantics=("arbitrary", "arbitrary"),
        ),
    )(logits, s_mat, mask_t)


def kernel(logits, word2label, label_words_mask):
    B, V = logits.shape
    C, A = word2label.shape
    assert C <= CP

    V_S = -(-V // 256) * 256
    vb = 2176 if V_S % 2176 == 0 else 128
    Bb = 256 if B % 256 == 0 else 8

    idx_t = jnp.zeros((A, CP), jnp.int32).at[:, :C].set(word2label.T)
    w_t = jnp.zeros((A, CP), jnp.float32).at[:, :C].set(
        label_words_mask.astype(jnp.float32).T
    )

    s_mat = _sc_build_s(idx_t.reshape(-1), w_t.reshape(-1), V_S=V_S, A=A)
    return _lse_mm(logits, s_mat, w_t, C=C, Bb=Bb, vb=vb)


# speedup vs baseline: 5.3384x; 2.8520x over previous
import functools

import jax
import jax.numpy as jnp
from jax import lax
from jax.experimental import pallas as pl
from jax.experimental.pallas import tpu as pltpu
from jax.experimental.pallas import tpu_sc as plsc

NC = 2
NS = 16
LANES = 16
NW = NC * NS
CP = 128
CLS_PER_W = CP // NW


@functools.partial(jax.jit, static_argnames=("A", "A_P"))
def _sc_pool(logits_t, idx_flat, w_flat, *, A, A_P):
    V, B = logits_t.shape
    n_ent = CP * A_P
    vec_b = B // LANES

    mesh = plsc.VectorSubcoreMesh(
        core_axis_name="c", subcore_axis_name="s", num_cores=NC, num_subcores=NS
    )

    @functools.partial(
        pl.kernel,
        mesh=mesh,
        compiler_params=pltpu.CompilerParams(needs_layout_passes=False),
        out_type=jax.ShapeDtypeStruct((CP, B), jnp.float32),
        scratch_types=[
            pltpu.VMEM((CP, A_P), jnp.int32),
            pltpu.VMEM((n_ent,), jnp.float32),
            pltpu.VMEM((A * LANES,), jnp.float32),
            pltpu.VMEM((A_P, B), jnp.float32),
            pltpu.VMEM((B,), jnp.float32),
            pltpu.SemaphoreType.DMA,
        ],
    )
    def pool(lt_hbm, idx_hbm, w_hbm, out_hbm, idx_v, w_v, ws_v, rows_v, acc_v,
             sem):
        wid = lax.axis_index("s") * NC + lax.axis_index("c")
        pltpu.sync_copy(idx_hbm, idx_v)
        pltpu.sync_copy(w_hbm, w_v)

        def cls_body(k, carry):
            c = wid * CLS_PER_W + k
            ent0 = c * A_P
            pltpu.async_copy(lt_hbm.at[idx_v.at[c]], rows_v, sem).wait()

            def splat(a, c2):
                ws_v[pl.ds(a * LANES, LANES)] = plsc.load_gather(
                    w_v, [jnp.full((LANES,), ent0 + a, jnp.int32)]
                )
                return c2

            lax.fori_loop(0, A, splat, 0)

            def col_body(q, c2):
                sl = pl.ds(q * LANES, LANES)

                def a_body(a, acc):
                    return acc + rows_v[a, sl] * ws_v[pl.ds(a * LANES, LANES)]

                acc_v[sl] = lax.fori_loop(
                    0, A, a_body, jnp.zeros((LANES,), jnp.float32)
                )
                return c2

            lax.fori_loop(0, vec_b, col_body, 0)
            pltpu.sync_copy(acc_v, out_hbm.at[c])
            return carry

        lax.fori_loop(0, CLS_PER_W, cls_body, 0)

    return pool(logits_t, idx_flat, w_flat)


def _lse_body(x_ref, out_ref, m_ref, sum_ref, *, nv):
    j = pl.program_id(0)
    i = pl.program_id(1)

    x = x_ref[...]
    bm = jnp.max(x, axis=0, keepdims=True)

    @pl.when(j == 0)
    def _init():
        m_ref[i] = bm
        sum_ref[i] = jnp.sum(jnp.exp(x - bm), axis=0, keepdims=True)

    @pl.when(j > 0)
    def _accum():
        m_old = m_ref[i]
        m_new = jnp.maximum(m_old, bm)
        alpha = jnp.where(m_old == m_new, 1.0, jnp.exp(m_old - m_new))
        e = jnp.exp(x - m_new)
        sum_ref[i] = sum_ref[i] * alpha + jnp.sum(e, axis=0, keepdims=True)
        m_ref[i] = m_new

    @pl.when(j == nv - 1)
    def _finalize():
        out_ref[...] = m_ref[i] + jnp.log(sum_ref[i])


@functools.partial(jax.jit, static_argnames=("Bb", "vb"))
def _lse(logits_t, *, Bb, vb):
    V, B = logits_t.shape
    nb = B // Bb
    nv = V // vb

    return pl.pallas_call(
        functools.partial(_lse_body, nv=nv),
        grid=(nv, nb),
        in_specs=[pl.BlockSpec((vb, Bb), lambda j, i: (j, i))],
        out_specs=pl.BlockSpec((1, Bb), lambda j, i: (0, i)),
        out_shape=jax.ShapeDtypeStruct((1, B), jnp.float32),
        scratch_shapes=[
            pltpu.VMEM((nb, 1, Bb), jnp.float32),
            pltpu.VMEM((nb, 1, Bb), jnp.float32),
        ],
        compiler_params=pltpu.CompilerParams(
            dimension_semantics=("arbitrary", "arbitrary"),
        ),
    )(logits_t)


def _combine_body(pooled_ref, norm_ref, mask_ref, out_ref, *, C):
    mask = mask_ref[...]
    summask = jnp.sum(mask, axis=1, keepdims=True)
    denom = jnp.clip(summask, 1e-9, None)
    inv = 1.0 / denom
    scale = summask * inv
    res = pooled_ref[...] * inv - scale * norm_ref[...]
    out_ref[...] = res[:C, :]


@functools.partial(jax.jit, static_argnames=("C", "Bb"))
def _combine(pooled, norm, mask_cp, *, C, Bb):
    _, B = pooled.shape
    nb = B // Bb

    return pl.pallas_call(
        functools.partial(_combine_body, C=C),
        grid=(nb,),
        in_specs=[
            pl.BlockSpec((CP, Bb), lambda i: (0, i)),
            pl.BlockSpec((1, Bb), lambda i: (0, i)),
            pl.BlockSpec(mask_cp.shape, lambda i: (0, 0)),
        ],
        out_specs=pl.BlockSpec((C, Bb), lambda i: (0, i)),
        out_shape=jax.ShapeDtypeStruct((C, B), jnp.float32),
    )(pooled, norm, mask_cp)


def kernel(logits, word2label, label_words_mask):
    B, V = logits.shape
    C, A = word2label.shape
    assert C <= CP and B % (LANES * NW) == 0

    A_P = -(-A // 8) * 8
    idx_cp = jnp.zeros((CP, A_P), jnp.int32).at[:C, :A].set(word2label)
    w_cp = jnp.zeros((CP, A_P), jnp.float32).at[:C, :A].set(
        label_words_mask.astype(jnp.float32)
    )
    mask_cp = w_cp[:, :A]

    lt = logits.T
    pooled = _sc_pool(lt, idx_cp, w_cp.reshape(-1), A=A, A_P=A_P)

    vb = next(d for d in (2000, 1600, 1000, 800, 500, 400, 250, 200, 100, 50,
                          25, 20, 10, 8, 5, 4, 2, 1) if V % d == 0)
    Bb = 512 if B % 512 == 0 else B
    norm = _lse(lt, Bb=Bb, vb=vb)

    out_t = _combine(pooled, norm, mask_cp, C=C, Bb=Bb)
    return out_t.T
